# baseline (speedup 1.0000x reference)
import jax
import jax.numpy as jnp
from jax import lax
from jax.experimental import pallas as pl
from jax.experimental.pallas import tpu as pltpu
from jax.experimental.pallas import tpu_sc as plsc

B, L, D = 16, 4096, 1024
NC, NS = 2, 16
LANES = 16
JV = D // LANES
JH = JV // 2
CH = 32
QUANT = 4 * CH
IDX_SIZE = L + QUANT
COMP_ITERS = L // LANES


def _pool_body(x_hbm, m_hbm, o_hbm, mask_v, idx_v, buf_a, buf_b,
               acc_v, tmp_v, out_v, shared_sp, sem_a, sem_b):
    c = lax.axis_index("c")
    s = lax.axis_index("s")
    b = c * (B // NC) + s // 2
    p = s % 2
    base_row = b * L

    pltpu.sync_copy(m_hbm.at[pl.ds(b * L, L)], mask_v)

    zero = jnp.zeros((LANES,), jnp.float32)
    for j in range(JV):
        acc_v[pl.ds(j * LANES, LANES)] = zero
        buf_b[CH - 1, pl.ds(j * LANES, LANES)] = zero

    lane_iota = lax.iota(jnp.int32, LANES)
    zero_i = jnp.zeros((LANES,), jnp.int32)

    def comp_body(i, cnt):
        mvec = mask_v[pl.ds(i * LANES, LANES)]
        ones = jnp.where(mvec != 0, 1, 0)
        incl = ones
        for d in (1, 2, 4, 8):
            incl = incl + jnp.where(
                lane_iota >= d, incl[jnp.maximum(lane_iota - d, 0)], 0)
        pos = zero_i
        for d in (8, 4, 2, 1):
            t = pos + d
            pos = jnp.where(incl[t - 1] <= lane_iota, t, pos)
        g = jnp.minimum(pos, LANES - 1)
        fidx = base_row + i * LANES + lane_iota
        idx_v[pl.ds(cnt, LANES)] = fidx[g]
        return cnt + incl[LANES - 1]

    cnt = lax.fori_loop(0, COMP_ITERS, comp_body, jnp.int32(0))

    pad_vec = zero_i + base_row
    for t in range(QUANT // LANES):
        idx_v[pl.ds(cnt + t * LANES, LANES)] = pad_vec

    padded = (cnt + QUANT - 1) // QUANT * QUANT
    half = padded // 2
    start = p * half
    myreal = jnp.clip(cnt - start, 0, half)
    mypad = half - myreal
    npairs = half // (2 * CH)

    def dma(g, buf, sem):
        off = pl.multiple_of(start + g * CH, CH)
        return pltpu.make_async_copy(
            x_hbm.at[idx_v.at[pl.ds(off, CH)]], buf, sem)

    def accumulate(buf):
        for k in range(2):
            def row_body(r, accs):
                return tuple(
                    a + buf[r, pl.ds(k * (D // 2) + j * LANES, LANES)]
                    for j, a in enumerate(accs))
            accs = lax.fori_loop(0, CH, row_body, tuple(zero for _ in range(JH)))
            for j in range(JH):
                plsc.addupdate(
                    acc_v.at[pl.ds(k * (D // 2) + j * LANES, LANES)], accs[j])

    @pl.when(npairs > 0)
    def _():
        dma(0, buf_a, sem_a).start()

    def pair_body(pp, carry):
        dma(2 * pp + 1, buf_b, sem_b).start()
        dma(2 * pp, buf_a, sem_a).wait()
        accumulate(buf_a)

        @pl.when(pp + 1 < npairs)
        def _():
            dma(2 * pp + 2, buf_a, sem_a).start()

        dma(2 * pp + 1, buf_b, sem_b).wait()
        accumulate(buf_b)
        return carry

    lax.fori_loop(0, npairs, pair_body, jnp.int32(0))

    mypad_v = zero + mypad.astype(jnp.float32)
    for j in range(JV):
        sl = pl.ds(j * LANES, LANES)
        acc_v[sl] = acc_v[sl] - mypad_v * buf_b[CH - 1, sl]

    @pl.when(p == 1)
    def _():
        pltpu.sync_copy(acc_v, shared_sp.at[s])

    plsc.subcore_barrier()

    @pl.when(p == 0)
    def _():
        pltpu.sync_copy(shared_sp.at[s + 1], tmp_v)
        cnt_v = zero + cnt.astype(jnp.float32)
        inv_v = 1.0 / jnp.maximum(cnt_v, 1.0)
        for j in range(JV):
            sl = pl.ds(j * LANES, LANES)
            out_v[sl] = (acc_v[sl] + tmp_v[sl]) * inv_v
        pltpu.sync_copy(out_v, o_hbm.at[pl.ds(b * D, D)])


_sc_pool = pl.kernel(
    _pool_body,
    out_type=jax.ShapeDtypeStruct((B * D,), jnp.float32),
    mesh=plsc.VectorSubcoreMesh(core_axis_name="c", subcore_axis_name="s"),
    scratch_types=[
        pltpu.VMEM((L,), jnp.int32),
        pltpu.VMEM((IDX_SIZE,), jnp.int32),
        pltpu.VMEM((CH, D), jnp.float32),
        pltpu.VMEM((CH, D), jnp.float32),
        pltpu.VMEM((D,), jnp.float32),
        pltpu.VMEM((D,), jnp.float32),
        pltpu.VMEM((D,), jnp.float32),
        pltpu.VMEM_SHARED((NS, D), jnp.float32),
        pltpu.SemaphoreType.DMA,
        pltpu.SemaphoreType.DMA,
    ],
)


def kernel(x, attention_mask):
    x2 = x.reshape(B * L, D)
    mflat = attention_mask.reshape(B * L)
    out = _sc_pool(x2, mflat)
    return out.reshape(B, D)


# speedup vs baseline: 1.0008x; 1.0008x over previous
import jax
import jax.numpy as jnp
from jax import lax
from jax.experimental import pallas as pl
from jax.experimental.pallas import tpu as pltpu
from jax.experimental.pallas import tpu_sc as plsc

B, L, D = 16, 4096, 1024
NC, NS = 2, 16
LANES = 16
JV = D // LANES
JH = JV // 2
CH = 32
QUANT = 4 * CH
IDX_SIZE = L + QUANT
COMP_ITERS = L // LANES


def _pool_body(x_hbm, m_hbm, o_hbm, mask_v, idx_v, buf_a, buf_b,
               acc_v, tmp_v, out_v, shared_sp, sem_a, sem_b):
    c = lax.axis_index("c")
    s = lax.axis_index("s")
    b = c * (B // NC) + s // 2
    p = s % 2
    base_row = b * L

    pltpu.sync_copy(m_hbm.at[pl.ds(b * L, L)], mask_v)

    zero = jnp.zeros((LANES,), jnp.float32)
    for j in range(JV):
        acc_v[pl.ds(j * LANES, LANES)] = zero
        buf_b[CH - 1, pl.ds(j * LANES, LANES)] = zero

    lane_iota = lax.iota(jnp.int32, LANES)
    zero_i = jnp.zeros((LANES,), jnp.int32)

    def comp_body(i, cnt):
        mvec = mask_v[pl.ds(i * LANES, LANES)]
        ones = jnp.where(mvec != 0, 1, 0)
        incl = ones
        for d in (1, 2, 4, 8):
            incl = incl + jnp.where(
                lane_iota >= d, incl[jnp.maximum(lane_iota - d, 0)], 0)
        pos = zero_i
        for d in (8, 4, 2, 1):
            t = pos + d
            pos = jnp.where(incl[t - 1] <= lane_iota, t, pos)
        g = jnp.minimum(pos, LANES - 1)
        fidx = base_row + i * LANES + lane_iota
        idx_v[pl.ds(cnt, LANES)] = fidx[g]
        return cnt + incl[LANES - 1]

    with jax.named_scope("comp_phase"):
        cnt = lax.fori_loop(0, COMP_ITERS, comp_body, jnp.int32(0))

    pad_vec = zero_i + base_row
    for t in range(QUANT // LANES):
        idx_v[pl.ds(cnt + t * LANES, LANES)] = pad_vec

    padded = (cnt + QUANT - 1) // QUANT * QUANT
    half = padded // 2
    start = p * half
    myreal = jnp.clip(cnt - start, 0, half)
    mypad = half - myreal
    npairs = half // (2 * CH)

    def dma(g, buf, sem):
        off = pl.multiple_of(start + g * CH, CH)
        return pltpu.make_async_copy(
            x_hbm.at[idx_v.at[pl.ds(off, CH)]], buf, sem)

    def accumulate(buf):
        for k in range(2):
            def row_body(r, accs):
                return tuple(
                    a + buf[r, pl.ds(k * (D // 2) + j * LANES, LANES)]
                    for j, a in enumerate(accs))
            accs = lax.fori_loop(0, CH, row_body, tuple(zero for _ in range(JH)))
            for j in range(JH):
                plsc.addupdate(
                    acc_v.at[pl.ds(k * (D // 2) + j * LANES, LANES)], accs[j])

    @pl.when(npairs > 0)
    def _():
        dma(0, buf_a, sem_a).start()

    def pair_body(pp, carry):
        dma(2 * pp + 1, buf_b, sem_b).start()
        dma(2 * pp, buf_a, sem_a).wait()
        accumulate(buf_a)

        @pl.when(pp + 1 < npairs)
        def _():
            dma(2 * pp + 2, buf_a, sem_a).start()

        dma(2 * pp + 1, buf_b, sem_b).wait()
        accumulate(buf_b)
        return carry

    with jax.named_scope("gather_phase"):
        lax.fori_loop(0, npairs, pair_body, jnp.int32(0))

    mypad_v = zero + mypad.astype(jnp.float32)
    for j in range(JV):
        sl = pl.ds(j * LANES, LANES)
        acc_v[sl] = acc_v[sl] - mypad_v * buf_b[CH - 1, sl]

    @pl.when(p == 1)
    def _():
        pltpu.sync_copy(acc_v, shared_sp.at[s])

    plsc.subcore_barrier()

    @pl.when(p == 0)
    def _():
        pltpu.sync_copy(shared_sp.at[s + 1], tmp_v)
        cnt_v = zero + cnt.astype(jnp.float32)
        inv_v = 1.0 / jnp.maximum(cnt_v, 1.0)
        for j in range(JV):
            sl = pl.ds(j * LANES, LANES)
            out_v[sl] = (acc_v[sl] + tmp_v[sl]) * inv_v
        pltpu.sync_copy(out_v, o_hbm.at[pl.ds(b * D, D)])


_sc_pool = pl.kernel(
    _pool_body,
    out_type=jax.ShapeDtypeStruct((B * D,), jnp.float32),
    mesh=plsc.VectorSubcoreMesh(core_axis_name="c", subcore_axis_name="s"),
    scratch_types=[
        pltpu.VMEM((L,), jnp.int32),
        pltpu.VMEM((IDX_SIZE,), jnp.int32),
        pltpu.VMEM((CH, D), jnp.float32),
        pltpu.VMEM((CH, D), jnp.float32),
        pltpu.VMEM((D,), jnp.float32),
        pltpu.VMEM((D,), jnp.float32),
        pltpu.VMEM((D,), jnp.float32),
        pltpu.VMEM_SHARED((NS, D), jnp.float32),
        pltpu.SemaphoreType.DMA,
        pltpu.SemaphoreType.DMA,
    ],
)


def kernel(x, attention_mask):
    x2 = x.reshape(B * L, D)
    mflat = attention_mask.reshape(B * L)
    out = _sc_pool(x2, mflat)
    return out.reshape(B, D)
